# baseline (speedup 1.0000x reference)
import functools

import jax
import jax.numpy as jnp
from jax.experimental import pallas as pl
from jax.experimental.pallas import tpu as pltpu

BATCH = 32
SEQ = 512
IN_DIM = 128
HID = 64
HEADS = 4
NUM_CLASSES = 4
NEG_SLOPE = 0.2
FEAT = HEADS * HID

G = 16
ROWS = G * SEQ


def _leaky(v):
    return jnp.maximum(v, NEG_SLOPE * v)


def _logit_mat(a_col):
    k = jax.lax.broadcasted_iota(jnp.int32, (FEAT, 128), 0)
    j = jax.lax.broadcasted_iota(jnp.int32, (FEAT, 128), 1)
    return jnp.where(j == k // HID, a_col, 0.0)


def _bcast_mat(dtype):
    p = jax.lax.broadcasted_iota(jnp.int32, (128, FEAT), 0)
    j = jax.lax.broadcasted_iota(jnp.int32, (128, FEAT), 1)
    return (p == j // HID).astype(dtype)


def _gat_messages(h, a_src, a_dst, bcast, b_row, valid_l, valid_r):
    R = h.shape[0]
    s_src = jnp.dot(h, a_src, preferred_element_type=jnp.float32)
    s_dst = jnp.dot(h, a_dst, preferred_element_type=jnp.float32)
    znar = jnp.zeros((1, 128), dtype=h.dtype)
    s_src_prev = jnp.concatenate([znar, s_src[: R - 1]], axis=0)
    s_src_next = jnp.concatenate([s_src[1:], znar], axis=0)
    e_l = _leaky(s_src_prev + s_dst)
    e_r = _leaky(s_src_next + s_dst)
    al = jax.nn.sigmoid(e_l - e_r)
    al = jnp.where(valid_r, al, 1.0)
    al = jnp.where(valid_l, al, 0.0)
    al = jnp.concatenate(
        [jnp.broadcast_to(al[:, k : k + 1], (R, HID)) for k in range(HEADS)],
        axis=1)
    zrow = jnp.zeros((1, FEAT), dtype=h.dtype)
    h_prev = jnp.concatenate([zrow, h[: R - 1]], axis=0)
    h_next = jnp.concatenate([h[1:], zrow], axis=0)
    return h_next + al * (h_prev - h_next) + b_row


def _fwd(x_ref, w0_ref, acat_ref, b0_ref,
         w1_ref, b1_ref,
         attw_ref, fcw_ref, fcb_ref, out_ref):
    x = x_ref[...]
    pos = jax.lax.broadcasted_iota(jnp.int32, (ROWS, 128), 0) % SEQ
    valid_l = pos != 0
    valid_r = pos != (SEQ - 1)
    bcast = _bcast_mat(x.dtype)
    acat = acat_ref[...]
    as0 = _logit_mat(acat[:, 0:1])
    ad0 = _logit_mat(acat[:, 1:2])
    as1 = _logit_mat(acat[:, 2:3])
    ad1 = _logit_mat(acat[:, 3:4])
    b0_row = b0_ref[...].reshape(1, FEAT)
    b1_row = b1_ref[...].reshape(1, FEAT)

    h = jnp.dot(x, w0_ref[...], preferred_element_type=jnp.float32)
    h = _gat_messages(h, as0, ad0, bcast, b0_row, valid_l, valid_r)
    h = jnp.maximum(h, 0.0)

    h = jnp.dot(h, w1_ref[...], preferred_element_type=jnp.float32)
    h = _gat_messages(h, as1, ad1, bcast, b1_row, valid_l, valid_r)
    h = jnp.maximum(h, 0.0)

    scores = jnp.dot(h, attw_ref[...],
                     preferred_element_type=jnp.float32)
    fcw = fcw_ref[...]
    fcb = fcb_ref[...].reshape(1, NUM_CLASSES)
    for s in range(G):
        sl = slice(s * SEQ, (s + 1) * SEQ)
        sc = scores[sl]
        w = jnp.exp(sc - jnp.max(sc))
        w = w / jnp.sum(w)
        pooled = jnp.sum(h[sl] * w, axis=0, keepdims=True)
        out_ref[s : s + 1, :] = (
            jnp.dot(pooled, fcw, preferred_element_type=jnp.float32) + fcb
        )


@functools.partial(jax.jit, static_argnames=())
def kernel(x, W0, a_src0, a_dst0, b0, W1, a_src1, a_dst1, b1,
           att_w, att_b, fc_w, fc_b):
    xf = x.reshape(BATCH * SEQ, IN_DIM)
    full = lambda shape: pl.BlockSpec(shape, lambda i: tuple(0 for _ in shape))
    return pl.pallas_call(
        _fwd,
        grid=(BATCH // G,),
        in_specs=[
            pl.BlockSpec((ROWS, IN_DIM), lambda i: (i, 0)),
            full((IN_DIM, FEAT)),
            full((FEAT, 4)), full((FEAT,)),
            full((FEAT, FEAT)), full((FEAT,)),
            full((FEAT, 1)),
            full((FEAT, NUM_CLASSES)),
            full((NUM_CLASSES,)),
        ],
        out_specs=pl.BlockSpec((G, NUM_CLASSES), lambda i: (i, 0)),
        out_shape=jax.ShapeDtypeStruct((BATCH, NUM_CLASSES), jnp.float32),
    )(xf, W0,
      jnp.concatenate([a_src0.reshape(FEAT, 1), a_dst0.reshape(FEAT, 1),
                       a_src1.reshape(FEAT, 1), a_dst1.reshape(FEAT, 1)],
                      axis=1),
      b0, W1, b1, att_w, fc_w, fc_b)


# speedup vs baseline: 1.2694x; 1.2694x over previous
import functools

import jax
import jax.numpy as jnp
from jax.experimental import pallas as pl
from jax.experimental.pallas import tpu as pltpu

BATCH = 32
SEQ = 512
IN_DIM = 128
HID = 64
HEADS = 4
NUM_CLASSES = 4
NEG_SLOPE = 0.2
FEAT = HEADS * HID

G = 16
ROWS = G * SEQ


def _leaky(v):
    return jnp.maximum(v, NEG_SLOPE * v)


def _logit_mat(a_col):
    k = jax.lax.broadcasted_iota(jnp.int32, (FEAT, 128), 0)
    j = jax.lax.broadcasted_iota(jnp.int32, (FEAT, 128), 1)
    return jnp.where(j == k // HID, a_col, 0.0)


def _bcast_mat(dtype):
    p = jax.lax.broadcasted_iota(jnp.int32, (128, FEAT), 0)
    j = jax.lax.broadcasted_iota(jnp.int32, (128, FEAT), 1)
    return (p == j // HID).astype(dtype)


def _gat_messages(h, a_src, a_dst, bcast, b_row, valid_l, valid_r):
    R = h.shape[0]
    s_src = jnp.dot(h, a_src, preferred_element_type=jnp.float32)
    s_dst = jnp.dot(h, a_dst, preferred_element_type=jnp.float32)
    znar = jnp.zeros((1, 128), dtype=h.dtype)
    s_src_prev = jnp.concatenate([znar, s_src[: R - 1]], axis=0)
    s_src_next = jnp.concatenate([s_src[1:], znar], axis=0)
    e_l = _leaky(s_src_prev + s_dst)
    e_r = _leaky(s_src_next + s_dst)
    al = jax.nn.sigmoid(e_l - e_r)
    al = jnp.where(valid_r, al, 1.0)
    al = jnp.where(valid_l, al, 0.0)
    al = jnp.dot(al, bcast, preferred_element_type=jnp.float32)
    zrow = jnp.zeros((1, FEAT), dtype=h.dtype)
    h_prev = jnp.concatenate([zrow, h[: R - 1]], axis=0)
    h_next = jnp.concatenate([h[1:], zrow], axis=0)
    return h_next + al * (h_prev - h_next) + b_row


def _fwd(x_ref, w0_ref, acat_ref, b0_ref,
         w1_ref, b1_ref,
         attw_ref, fcw_ref, fcb_ref, out_ref):
    x = x_ref[...]
    pos = jax.lax.broadcasted_iota(jnp.int32, (ROWS, 128), 0) % SEQ
    valid_l = pos != 0
    valid_r = pos != (SEQ - 1)
    bcast = _bcast_mat(x.dtype)
    acat = acat_ref[...]
    as0 = _logit_mat(acat[:, 0:1])
    ad0 = _logit_mat(acat[:, 1:2])
    as1 = _logit_mat(acat[:, 2:3])
    ad1 = _logit_mat(acat[:, 3:4])
    b0_row = b0_ref[...].reshape(1, FEAT)
    b1_row = b1_ref[...].reshape(1, FEAT)

    h = jnp.dot(x, w0_ref[...], preferred_element_type=jnp.float32)
    h = _gat_messages(h, as0, ad0, bcast, b0_row, valid_l, valid_r)
    h = jnp.maximum(h, 0.0)

    h = jnp.dot(h, w1_ref[...], preferred_element_type=jnp.float32)
    h = _gat_messages(h, as1, ad1, bcast, b1_row, valid_l, valid_r)
    h = jnp.maximum(h, 0.0)

    scores = jnp.dot(h, attw_ref[...],
                     preferred_element_type=jnp.float32)
    fcw = fcw_ref[...]
    fcb = fcb_ref[...].reshape(1, NUM_CLASSES)
    for s in range(G):
        sl = slice(s * SEQ, (s + 1) * SEQ)
        sc = scores[sl]
        w = jnp.exp(sc - jnp.max(sc))
        w = w / jnp.sum(w)
        pooled = jnp.sum(h[sl] * w, axis=0, keepdims=True)
        out_ref[s : s + 1, :] = (
            jnp.dot(pooled, fcw, preferred_element_type=jnp.float32) + fcb
        )


@functools.partial(jax.jit, static_argnames=())
def kernel(x, W0, a_src0, a_dst0, b0, W1, a_src1, a_dst1, b1,
           att_w, att_b, fc_w, fc_b):
    xf = x.reshape(BATCH * SEQ, IN_DIM)
    full = lambda shape: pl.BlockSpec(shape, lambda i: tuple(0 for _ in shape))
    return pl.pallas_call(
        _fwd,
        grid=(BATCH // G,),
        in_specs=[
            pl.BlockSpec((ROWS, IN_DIM), lambda i: (i, 0)),
            full((IN_DIM, FEAT)),
            full((FEAT, 4)), full((FEAT,)),
            full((FEAT, FEAT)), full((FEAT,)),
            full((FEAT, 1)),
            full((FEAT, NUM_CLASSES)),
            full((NUM_CLASSES,)),
        ],
        out_specs=pl.BlockSpec((G, NUM_CLASSES), lambda i: (i, 0)),
        out_shape=jax.ShapeDtypeStruct((BATCH, NUM_CLASSES), jnp.float32),
    )(xf, W0,
      jnp.concatenate([a_src0.reshape(FEAT, 1), a_dst0.reshape(FEAT, 1),
                       a_src1.reshape(FEAT, 1), a_dst1.reshape(FEAT, 1)],
                      axis=1),
      b0, W1, b1, att_w, fc_w, fc_b)


# speedup vs baseline: 1.2832x; 1.0109x over previous
import functools

import jax
import jax.numpy as jnp
from jax.experimental import pallas as pl
from jax.experimental.pallas import tpu as pltpu

BATCH = 32
SEQ = 512
IN_DIM = 128
HID = 64
HEADS = 4
NUM_CLASSES = 4
NEG_SLOPE = 0.2
FEAT = HEADS * HID

G = 16
ROWS = G * SEQ


def _leaky(v):
    return jnp.maximum(v, NEG_SLOPE * v)


def _logit_mat(a_col):
    k = jax.lax.broadcasted_iota(jnp.int32, (FEAT, 128), 0)
    j = jax.lax.broadcasted_iota(jnp.int32, (FEAT, 128), 1)
    return jnp.where(j == k // HID, a_col, 0.0)


def _bcast_mat(dtype):
    p = jax.lax.broadcasted_iota(jnp.int32, (128, FEAT), 0)
    j = jax.lax.broadcasted_iota(jnp.int32, (128, FEAT), 1)
    return (p == j // HID).astype(dtype)


def _gat_messages(h, s_src, s_dst, bcast, b_row, valid_l, valid_r):
    R = h.shape[0]
    znar = jnp.zeros((1, 128), dtype=h.dtype)
    s_src_prev = jnp.concatenate([znar, s_src[: R - 1]], axis=0)
    s_src_next = jnp.concatenate([s_src[1:], znar], axis=0)
    e_l = _leaky(s_src_prev + s_dst)
    e_r = _leaky(s_src_next + s_dst)
    al = jax.nn.sigmoid(e_l - e_r)
    al = jnp.where(valid_r, al, 1.0)
    al = jnp.where(valid_l, al, 0.0)
    al = jnp.dot(al, bcast, preferred_element_type=jnp.float32)
    zrow = jnp.zeros((1, FEAT), dtype=h.dtype)
    h_prev = jnp.concatenate([zrow, h[: R - 1]], axis=0)
    h_next = jnp.concatenate([h[1:], zrow], axis=0)
    return h_next + al * (h_prev - h_next) + b_row


def _fwd(x_ref, w0_ref, acat_ref, b0_ref,
         w1_ref, b1_ref,
         attw_ref, fcw_ref, fcb_ref, out_ref):
    x = x_ref[...]
    pos = jax.lax.broadcasted_iota(jnp.int32, (ROWS, 128), 0) % SEQ
    valid_l = pos != 0
    valid_r = pos != (SEQ - 1)
    bcast = _bcast_mat(x.dtype)
    acat = acat_ref[...]
    as0 = _logit_mat(acat[:, 0:1])
    ad0 = _logit_mat(acat[:, 1:2])
    as1 = _logit_mat(acat[:, 2:3])
    ad1 = _logit_mat(acat[:, 3:4])
    b0_row = b0_ref[...].reshape(1, FEAT)
    b1_row = b1_ref[...].reshape(1, FEAT)

    w0 = w0_ref[...]
    wa_s0 = jnp.dot(w0, as0, preferred_element_type=jnp.float32)
    wa_d0 = jnp.dot(w0, ad0, preferred_element_type=jnp.float32)
    h = jnp.dot(x, w0, preferred_element_type=jnp.float32)
    s_src = jnp.dot(x, wa_s0, preferred_element_type=jnp.float32)
    s_dst = jnp.dot(x, wa_d0, preferred_element_type=jnp.float32)
    h = _gat_messages(h, s_src, s_dst, bcast, b0_row, valid_l, valid_r)
    h = jnp.maximum(h, 0.0)

    h = jnp.dot(h, w1_ref[...], preferred_element_type=jnp.float32)
    s_src = jnp.dot(h, as1, preferred_element_type=jnp.float32)
    s_dst = jnp.dot(h, ad1, preferred_element_type=jnp.float32)
    h = _gat_messages(h, s_src, s_dst, bcast, b1_row, valid_l, valid_r)
    h = jnp.maximum(h, 0.0)

    scores = jnp.dot(h, attw_ref[...],
                     preferred_element_type=jnp.float32)
    fcw = fcw_ref[...]
    fcb = fcb_ref[...].reshape(1, NUM_CLASSES)
    for s in range(G):
        sl = slice(s * SEQ, (s + 1) * SEQ)
        sc = scores[sl]
        w = jnp.exp(sc - jnp.max(sc))
        w = w / jnp.sum(w)
        pooled = jnp.sum(h[sl] * w, axis=0, keepdims=True)
        out_ref[s : s + 1, :] = (
            jnp.dot(pooled, fcw, preferred_element_type=jnp.float32) + fcb
        )


@functools.partial(jax.jit, static_argnames=())
def kernel(x, W0, a_src0, a_dst0, b0, W1, a_src1, a_dst1, b1,
           att_w, att_b, fc_w, fc_b):
    xf = x.reshape(BATCH * SEQ, IN_DIM)
    full = lambda shape: pl.BlockSpec(shape, lambda i: tuple(0 for _ in shape))
    return pl.pallas_call(
        _fwd,
        grid=(BATCH // G,),
        in_specs=[
            pl.BlockSpec((ROWS, IN_DIM), lambda i: (i, 0)),
            full((IN_DIM, FEAT)),
            full((FEAT, 4)), full((FEAT,)),
            full((FEAT, FEAT)), full((FEAT,)),
            full((FEAT, 1)),
            full((FEAT, NUM_CLASSES)),
            full((NUM_CLASSES,)),
        ],
        out_specs=pl.BlockSpec((G, NUM_CLASSES), lambda i: (i, 0)),
        out_shape=jax.ShapeDtypeStruct((BATCH, NUM_CLASSES), jnp.float32),
    )(xf, W0,
      jnp.concatenate([a_src0.reshape(FEAT, 1), a_dst0.reshape(FEAT, 1),
                       a_src1.reshape(FEAT, 1), a_dst1.reshape(FEAT, 1)],
                      axis=1),
      b0, W1, b1, att_w, fc_w, fc_b)


# speedup vs baseline: 1.2889x; 1.0044x over previous
import functools

import jax
import jax.numpy as jnp
from jax.experimental import pallas as pl

BATCH = 32
SEQ = 512
IN_DIM = 128
HID = 64
HEADS = 4
NUM_CLASSES = 4
NEG_SLOPE = 0.2
FEAT = HEADS * HID

G = 16
ROWS = G * SEQ


def _leaky(v):
    return jnp.maximum(v, NEG_SLOPE * v)


def _logit_mat(a_col):
    k = jax.lax.broadcasted_iota(jnp.int32, (FEAT, 128), 0)
    j = jax.lax.broadcasted_iota(jnp.int32, (FEAT, 128), 1)
    return jnp.where(j == k // HID, a_col, 0.0)


def _bcast_mat(dtype):
    p = jax.lax.broadcasted_iota(jnp.int32, (128, FEAT), 0)
    j = jax.lax.broadcasted_iota(jnp.int32, (128, FEAT), 1)
    return (p == j // HID).astype(dtype)


def _gat_messages(h, s_src, s_dst, bcast, b_row, valid_l, valid_r):
    R = h.shape[0]
    znar = jnp.zeros((1, 128), dtype=h.dtype)
    s_src_prev = jnp.concatenate([znar, s_src[: R - 1]], axis=0)
    s_src_next = jnp.concatenate([s_src[1:], znar], axis=0)
    e_l = _leaky(s_src_prev + s_dst)
    e_r = _leaky(s_src_next + s_dst)
    al = jax.nn.sigmoid(e_l - e_r)
    al = jnp.where(valid_r, al, 1.0)
    al = jnp.where(valid_l, al, 0.0)
    al = jnp.dot(al, bcast, preferred_element_type=jnp.float32)
    zrow = jnp.zeros((1, FEAT), dtype=h.dtype)
    h_prev = jnp.concatenate([zrow, h[: R - 1]], axis=0)
    h_next = jnp.concatenate([h[1:], zrow], axis=0)
    return h_next + al * (h_prev - h_next) + b_row


def _fwd(x_ref, w0_ref, acat_ref, b0_ref,
         w1_ref, b1_ref,
         attw_ref, fcw_ref, fcb_ref, out_ref):
    x = x_ref[...]
    pos = jax.lax.broadcasted_iota(jnp.int32, (ROWS, 128), 0) % SEQ
    valid_l = pos != 0
    valid_r = pos != (SEQ - 1)
    bcast = _bcast_mat(x.dtype)
    acat = acat_ref[...]
    as0 = _logit_mat(acat[:, 0:1])
    ad0 = _logit_mat(acat[:, 1:2])
    as1 = _logit_mat(acat[:, 2:3])
    ad1 = _logit_mat(acat[:, 3:4])
    b0_row = b0_ref[...].reshape(1, FEAT)
    b1_row = b1_ref[...].reshape(1, FEAT)

    w0 = w0_ref[...]
    wa_s0 = jnp.dot(w0, as0, preferred_element_type=jnp.float32)
    wa_d0 = jnp.dot(w0, ad0, preferred_element_type=jnp.float32)
    h = jnp.dot(x, w0, preferred_element_type=jnp.float32)
    s_src = jnp.dot(x, wa_s0, preferred_element_type=jnp.float32)
    s_dst = jnp.dot(x, wa_d0, preferred_element_type=jnp.float32)
    h = _gat_messages(h, s_src, s_dst, bcast, b0_row, valid_l, valid_r)
    h = jnp.maximum(h, 0.0)

    h = jnp.dot(h, w1_ref[...], preferred_element_type=jnp.float32)
    s_src = jnp.dot(h, as1, preferred_element_type=jnp.float32)
    s_dst = jnp.dot(h, ad1, preferred_element_type=jnp.float32)
    h = _gat_messages(h, s_src, s_dst, bcast, b1_row, valid_l, valid_r)
    h = jnp.maximum(h, 0.0)

    scores = jnp.dot(h, attw_ref[...],
                     preferred_element_type=jnp.float32)
    fcw = fcw_ref[...]
    fcb = fcb_ref[...].reshape(1, NUM_CLASSES)
    for s in range(G):
        sl = slice(s * SEQ, (s + 1) * SEQ)
        sc = scores[sl]
        w = jnp.exp(sc - jnp.max(sc))
        w = w / jnp.sum(w)
        pooled = jnp.sum(h[sl] * w, axis=0, keepdims=True)
        out_ref[s : s + 1, :] = (
            jnp.dot(pooled, fcw, preferred_element_type=jnp.float32) + fcb
        )


@functools.partial(jax.jit, static_argnames=())
def kernel(x, W0, a_src0, a_dst0, b0, W1, a_src1, a_dst1, b1,
           att_w, att_b, fc_w, fc_b):
    xf = x.reshape(BATCH * SEQ, IN_DIM)
    full = lambda shape: pl.BlockSpec(shape, lambda i: tuple(0 for _ in shape))
    return pl.pallas_call(
        _fwd,
        grid=(BATCH // G,),
        in_specs=[
            pl.BlockSpec((ROWS, IN_DIM), lambda i: (i, 0)),
            full((IN_DIM, FEAT)),
            full((FEAT, 4)), full((FEAT,)),
            full((FEAT, FEAT)), full((FEAT,)),
            full((FEAT, 1)),
            full((FEAT, NUM_CLASSES)),
            full((NUM_CLASSES,)),
        ],
        out_specs=pl.BlockSpec((G, NUM_CLASSES), lambda i: (i, 0)),
        out_shape=jax.ShapeDtypeStruct((BATCH, NUM_CLASSES), jnp.float32),
    )(xf, W0,
      jnp.concatenate([a_src0.reshape(FEAT, 1), a_dst0.reshape(FEAT, 1),
                       a_src1.reshape(FEAT, 1), a_dst1.reshape(FEAT, 1)],
                      axis=1),
      b0, W1, b1, att_w, fc_w, fc_b)
